# baseline (speedup 1.0000x reference)
import jax
import jax.numpy as jnp
from jax.experimental import pallas as pl
from jax.experimental.pallas import tpu as pltpu

_BM = 200


def _gcn_body(x_ref, A_ref, W1_ref, b1_ref, W2_ref, b2_ref, out_ref,
              y1_s, y2_s):
    p = pl.program_id(0)
    i = pl.program_id(1)

    @pl.when(jnp.logical_and(p == 0, i == 0))
    def _():
        y1_s[...] = jnp.dot(x_ref[...], W1_ref[...],
                            preferred_element_type=jnp.float32)

    a = A_ref[...]

    @pl.when(p == 0)
    def _():
        z = jnp.dot(a, y1_s[...], preferred_element_type=jnp.float32)
        h = jnp.maximum(z + b1_ref[...], 0.0)
        y2_s[pl.ds(i * _BM, _BM), :] = jnp.dot(
            h, W2_ref[...], preferred_element_type=jnp.float32)

    @pl.when(p == 1)
    def _():
        z = jnp.dot(a, y2_s[...], preferred_element_type=jnp.float32)
        z = z + b2_ref[...]
        m = jnp.max(z, axis=1, keepdims=True)
        lse = jnp.log(jnp.sum(jnp.exp(z - m), axis=1, keepdims=True))
        out_ref[...] = z - m - lse


def kernel(x, A, W1, b1, W2, b2):
    n, d = x.shape
    nhid = W1.shape[1]
    nb = n // _BM
    return pl.pallas_call(
        _gcn_body,
        grid=(2, nb),
        in_specs=[
            pl.BlockSpec((n, d), lambda p, i: (0, 0)),
            pl.BlockSpec((_BM, n), lambda p, i: (i, 0)),
            pl.BlockSpec((d, nhid), lambda p, i: (0, 0)),
            pl.BlockSpec((1, nhid), lambda p, i: (0, 0)),
            pl.BlockSpec((nhid, d), lambda p, i: (0, 0)),
            pl.BlockSpec((1, d), lambda p, i: (0, 0)),
        ],
        out_specs=pl.BlockSpec((_BM, d), lambda p, i: (i, 0)),
        out_shape=jax.ShapeDtypeStruct((n, d), jnp.float32),
        scratch_shapes=[
            pltpu.VMEM((n, nhid), jnp.float32),
            pltpu.VMEM((n, d), jnp.float32),
        ],
        compiler_params=pltpu.CompilerParams(
            dimension_semantics=("arbitrary", "arbitrary"),
            vmem_limit_bytes=100 * 1024 * 1024,
        ),
    )(x, A, W1, b1.reshape(1, nhid), W2, b2.reshape(1, d))


# speedup vs baseline: 1.1248x; 1.1248x over previous
import jax
import jax.numpy as jnp
from jax.experimental import pallas as pl
from jax.experimental.pallas import tpu as pltpu

_BM = 400


def _gcn_body(x_ref, A_ref, W1_ref, b1_ref, W2_ref, b2_ref, out_ref,
              y1_s, y2_s):
    p = pl.program_id(0)
    i = pl.program_id(1)

    @pl.when(jnp.logical_and(p == 0, i == 0))
    def _():
        y1_s[...] = jnp.dot(x_ref[...], W1_ref[...],
                            preferred_element_type=jnp.float32)

    a = A_ref[...]

    @pl.when(p == 0)
    def _():
        z = jnp.dot(a, y1_s[...], preferred_element_type=jnp.float32)
        h = jnp.maximum(z + b1_ref[...], 0.0)
        y2_s[pl.ds(i * _BM, _BM), :] = jnp.dot(
            h, W2_ref[...], preferred_element_type=jnp.float32)

    @pl.when(p == 1)
    def _():
        z = jnp.dot(a, y2_s[...], preferred_element_type=jnp.float32)
        z = z + b2_ref[...]
        m = jnp.max(z, axis=1, keepdims=True)
        lse = jnp.log(jnp.sum(jnp.exp(z - m), axis=1, keepdims=True))
        out_ref[...] = z - m - lse


def kernel(x, A, W1, b1, W2, b2):
    n, d = x.shape
    nhid = W1.shape[1]
    nb = n // _BM
    return pl.pallas_call(
        _gcn_body,
        grid=(2, nb),
        in_specs=[
            pl.BlockSpec((n, d), lambda p, i: (0, 0)),
            pl.BlockSpec((_BM, n), lambda p, i: (i, 0)),
            pl.BlockSpec((d, nhid), lambda p, i: (0, 0)),
            pl.BlockSpec((1, nhid), lambda p, i: (0, 0)),
            pl.BlockSpec((nhid, d), lambda p, i: (0, 0)),
            pl.BlockSpec((1, d), lambda p, i: (0, 0)),
        ],
        out_specs=pl.BlockSpec((_BM, d), lambda p, i: (i, 0)),
        out_shape=jax.ShapeDtypeStruct((n, d), jnp.float32),
        scratch_shapes=[
            pltpu.VMEM((n, nhid), jnp.float32),
            pltpu.VMEM((n, d), jnp.float32),
        ],
        compiler_params=pltpu.CompilerParams(
            dimension_semantics=("arbitrary", "arbitrary"),
            vmem_limit_bytes=100 * 1024 * 1024,
        ),
    )(x, A, W1, b1.reshape(1, nhid), W2, b2.reshape(1, d))
